# baseline (speedup 1.0000x reference)
import functools

import jax
import jax.numpy as jnp
from jax import lax
from jax.experimental import pallas as pl
from jax.experimental.pallas import tpu as pltpu
from jax.experimental.pallas import tpu_sc as plsc

_NC = 2
_NS = 16
_NW = _NC * _NS

_V = 1000000
_G = _V // 4
_NCOLS = _V // 128
_D = 32


def _shuffle_block(src, dst, n_groups):
  def body(gl, carry):
    for k in range(8):
      c_vec = lax.iota(jnp.int32, 16) + (k % 2) * 16
      lane = jnp.full((16,), 4 * gl + (k // 2), jnp.int32)
      dst[gl, pl.ds(16 * k, 16)] = plsc.load_gather(src, [c_vec, lane])
    return carry
  lax.fori_loop(0, n_groups, body, 0, unroll=4)


@functools.lru_cache(maxsize=None)
def _make_relayout():
  mesh = plsc.VectorSubcoreMesh(core_axis_name="c", subcore_axis_name="s")

  @functools.partial(
      pl.kernel,
      mesh=mesh,
      out_type=jax.ShapeDtypeStruct((_G, 128), jnp.float32),
      scratch_types=[
          [pltpu.VMEM((32, 128), jnp.float32) for _ in range(2)],
          [pltpu.VMEM((32, 128), jnp.float32) for _ in range(2)],
          [pltpu.SemaphoreType.DMA for _ in range(2)],
          [pltpu.SemaphoreType.DMA for _ in range(2)],
          pltpu.VMEM((32, 64), jnp.float32),
          pltpu.VMEM((16, 128), jnp.float32),
          pltpu.SemaphoreType.DMA,
      ],
      compiler_params=pltpu.CompilerParams(needs_layout_passes=False),
  )
  def k(tabT_hbm, grp_hbm, ibufs, obufs, isems, osems, tin, tout, tsem):
    wid = lax.axis_index("s") * _NC + lax.axis_index("c")
    q, r = divmod(_NCOLS, _NW)
    lo = wid * q + jnp.minimum(wid, r)
    n = q + jnp.where(wid < r, 1, 0)

    def in_copy(i, bb):
      return pltpu.make_async_copy(
          tabT_hbm.at[:, pl.ds((lo + i) * 128, 128)], ibufs[bb], isems[bb])

    in_copy(0, 0).start()

    def body(i, carry):
      b = lax.rem(i, 2)
      for bb in range(2):
        @pl.when(b == bb)
        def _():
          @pl.when(i + 1 < n)
          def _():
            in_copy(i + 1, 1 - bb).start()
          in_copy(i, bb).wait()
          @pl.when(i >= 2)
          def _():
            pltpu.make_async_copy(
                obufs[bb], grp_hbm.at[pl.ds(lo * 32, 32)], osems[bb]).wait()
          _shuffle_block(ibufs[bb], obufs[bb], 32)
          pltpu.async_copy(
              obufs[bb], grp_hbm.at[pl.ds((lo + i) * 32, 32)], osems[bb])
      return carry

    lax.fori_loop(0, n, body, 0)
    for bb in range(2):
      pltpu.make_async_copy(
          obufs[bb], grp_hbm.at[pl.ds(lo * 32, 32)], osems[bb]).wait()

    @pl.when(wid == 0)
    def _():
      pltpu.sync_copy(tabT_hbm.at[:, pl.ds(_NCOLS * 128, 64)], tin)
      def tbody(gl, carry):
        for k in range(8):
          c_vec = lax.iota(jnp.int32, 16) + (k % 2) * 16
          lane = jnp.full((16,), 4 * gl + (k // 2), jnp.int32)
          tout[gl, pl.ds(16 * k, 16)] = plsc.load_gather(tin, [c_vec, lane])
        return carry
      lax.fori_loop(0, 16, tbody, 0)
      pltpu.async_copy(tout, grp_hbm.at[pl.ds(_NCOLS * 32, 16)], tsem).wait()

  return k


@functools.lru_cache(maxsize=None)
def _make_gather(R, S):
  i_per_w = R // _NW
  j_per_w = i_per_w * S
  mesh = plsc.VectorSubcoreMesh(core_axis_name="c", subcore_axis_name="s")

  @functools.partial(
      pl.kernel,
      mesh=mesh,
      out_type=jax.ShapeDtypeStruct((S, _D, R), jnp.float32),
      scratch_types=[
          pltpu.VMEM((j_per_w,), jnp.int32),
          [pltpu.VMEM((i_per_w,), jnp.int32) for _ in range(2)],
          [pltpu.VMEM((i_per_w,), jnp.int32) for _ in range(2)],
          [pltpu.VMEM((i_per_w, 128), jnp.float32) for _ in range(2)],
          [pltpu.VMEM((_D, i_per_w), jnp.float32) for _ in range(2)],
          [pltpu.SemaphoreType.DMA for _ in range(2)],
          [pltpu.SemaphoreType.DMA for _ in range(2)],
      ],
      compiler_params=pltpu.CompilerParams(needs_layout_passes=False),
  )
  def k(idx_hbm, grp_hbm, outT_hbm, idxb, idxg, pvec, gbufs, obufs,
        gsems, osems):
    wid = lax.axis_index("s") * _NC + lax.axis_index("c")
    base_i = wid * i_per_w
    pltpu.sync_copy(idx_hbm.at[pl.ds(wid * j_per_w, j_per_w)], idxb)

    def start_gather(jj, bb):
      for m in range(i_per_w // 16):
        jloc = (lax.iota(jnp.int32, 16) + 16 * m) * S + jj
        rv = plsc.load_gather(idxb, [jloc])
        idxg[bb][pl.ds(16 * m, 16)] = lax.shift_right_logical(rv, 2)
        pvec[bb][pl.ds(16 * m, 16)] = lax.shift_left(
            lax.bitwise_and(rv, 3), 5)
      pltpu.async_copy(grp_hbm.at[idxg[bb]], gbufs[bb], gsems[bb])

    start_gather(0, 0)

    def body(jj, carry):
      b = lax.rem(jj, 2)
      for bb in range(2):
        @pl.when(b == bb)
        def _():
          @pl.when(jj + 1 < S)
          def _():
            start_gather(jj + 1, 1 - bb)
          pltpu.make_async_copy(
              grp_hbm.at[idxg[bb]], gbufs[bb], gsems[bb]).wait()
          @pl.when(jj >= 2)
          def _():
            pltpu.make_async_copy(
                obufs[bb], outT_hbm.at[0, :, pl.ds(base_i, i_per_w)],
                osems[bb]).wait()
          def cbody(c, carry2):
            for m in range(i_per_w // 16):
              il = lax.iota(jnp.int32, 16) + 16 * m
              pv = pvec[bb][pl.ds(16 * m, 16)] + c
              obufs[bb][c, pl.ds(16 * m, 16)] = plsc.load_gather(
                  gbufs[bb], [il, pv])
            return carry2
          lax.fori_loop(0, _D, cbody, 0, unroll=4)
          pltpu.async_copy(
              obufs[bb], outT_hbm.at[jj, :, pl.ds(base_i, i_per_w)],
              osems[bb])
      return carry

    lax.fori_loop(0, S, body, 0)
    for bb in range(2):
      pltpu.make_async_copy(
          obufs[bb], outT_hbm.at[0, :, pl.ds(base_i, i_per_w)],
          osems[bb]).wait()

  return k


def kernel(indices, table):
  R, S = indices.shape
  idx1d = indices.reshape(-1).astype(jnp.int32)
  tabT = table.T
  grp = _make_relayout()(tabT)
  outT = _make_gather(R, S)(idx1d, grp)
  return outT.transpose(2, 0, 1)


# speedup vs baseline: 1.5001x; 1.5001x over previous
import functools

import jax
import jax.numpy as jnp
from jax import lax
from jax.experimental import pallas as pl
from jax.experimental.pallas import tpu as pltpu
from jax.experimental.pallas import tpu_sc as plsc

_NC = 2
_NS = 16
_NW = _NC * _NS
_D = 32
_RD = 4


@functools.lru_cache(maxsize=None)
def _make_gather(R, S, G):
  i_per_w = R // _NW
  j_per_w = i_per_w * S
  nm = i_per_w // 16
  mesh = plsc.VectorSubcoreMesh(core_axis_name="c", subcore_axis_name="s")

  @functools.partial(
      pl.kernel,
      mesh=mesh,
      out_type=jax.ShapeDtypeStruct((S, _D, R), jnp.float32),
      scratch_types=[
          pltpu.VMEM((j_per_w,), jnp.int32),
          pltpu.VMEM((S, i_per_w), jnp.int32),
          pltpu.VMEM((S, i_per_w), jnp.int32),
          [pltpu.VMEM((i_per_w, 128), jnp.float32) for _ in range(_RD)],
          [pltpu.VMEM((_D, i_per_w), jnp.float32) for _ in range(2)],
          [pltpu.SemaphoreType.DMA for _ in range(_RD)],
          [pltpu.SemaphoreType.DMA for _ in range(2)],
      ],
      compiler_params=pltpu.CompilerParams(needs_layout_passes=False),
  )
  def k(idx_hbm, grp_hbm, outT_hbm, idxb, idxg, pvec, gbufs, obufs,
        gsems, osems):
    wid = lax.axis_index("s") * _NC + lax.axis_index("c")
    base_i = wid * i_per_w
    pltpu.sync_copy(idx_hbm.at[pl.ds(wid * j_per_w, j_per_w)], idxb)

    iota = lax.iota(jnp.int32, 16)
    jbase = [(iota + 16 * m) * S for m in range(nm)]
    ilvec = [iota + 16 * m for m in range(nm)]

    def prep(jj, carry):
      for m in range(nm):
        rv = plsc.load_gather(idxb, [jbase[m] + jj])
        idxg[jj, pl.ds(16 * m, 16)] = lax.shift_right_logical(rv, 2)
        pvec[jj, pl.ds(16 * m, 16)] = lax.shift_left(
            lax.bitwise_and(rv, 3), 5)
      return carry
    lax.fori_loop(0, S, prep, 0, unroll=4)

    def g_copy(jj, bb):
      return pltpu.make_async_copy(
          grp_hbm.at[idxg.at[jj]], gbufs[bb], gsems[bb])

    for jj in range(_RD - 1):
      g_copy(jj, jj % _RD).start()

    def body(jj, carry):
      b = lax.rem(jj, _RD)
      ob = lax.rem(jj, 2)
      @pl.when(jj + _RD - 1 < S)
      def _():
        for bb in range(_RD):
          @pl.when(lax.rem(jj + _RD - 1, _RD) == bb)
          def _():
            g_copy(jj + _RD - 1, bb).start()
      for bb in range(_RD):
        @pl.when(b == bb)
        def _():
          g_copy(jj, bb).wait()
      for ib in range(2):
        @pl.when(ob == ib)
        def _():
          @pl.when(jj >= 2)
          def _():
            pltpu.make_async_copy(
                obufs[ib], outT_hbm.at[0, :, pl.ds(base_i, i_per_w)],
                osems[ib]).wait()
          for bb in range(_RD):
            @pl.when(b == bb)
            def _():
              pv = [pvec[jj, pl.ds(16 * m, 16)] for m in range(nm)]
              def cbody(c, carry2):
                for m in range(nm):
                  obufs[ib][c, pl.ds(16 * m, 16)] = plsc.load_gather(
                      gbufs[bb], [ilvec[m], pv[m] + c])
                return carry2
              lax.fori_loop(0, _D, cbody, 0, unroll=8)
          pltpu.async_copy(
              obufs[ib], outT_hbm.at[jj, :, pl.ds(base_i, i_per_w)],
              osems[ib])
      return carry

    lax.fori_loop(0, S, body, 0)
    for ib in range(2):
      pltpu.make_async_copy(
          obufs[ib], outT_hbm.at[0, :, pl.ds(base_i, i_per_w)],
          osems[ib]).wait()

  return k


def kernel(indices, table):
  R, S = indices.shape
  V, D = table.shape
  G = V // 4
  idx1d = indices.reshape(-1).astype(jnp.int32)
  grp = table.reshape(G, 4 * D)
  outT = _make_gather(R, S, G)(idx1d, grp)
  return outT.transpose(2, 0, 1)
